# baseline (speedup 1.0000x reference)
import jax
import jax.numpy as jnp
from jax.experimental import pallas as pl

_EMBED = 256
_S = 8
_COLS_PER_BLOCK = 4096


def _embed_body(xv, xd, x0, x1, x2, Wv, Wd, Wp, bv, bd, bp, out):
    Bb, C = xv.shape
    Tb = C // _S
    xs = [x[...].astype(jnp.int8).reshape(Bb, Tb, _S)
          for x in (xv, xd, x0, x1, x2)]
    X = jnp.concatenate(xs, axis=2).astype(jnp.bfloat16)
    W = jnp.concatenate(
        [Wv[...], Wd[...], Wp[0], Wp[1], Wp[2]], axis=1).astype(jnp.bfloat16)
    dn = (((2,), (1,)), ((), ()))
    acc = jax.lax.dot_general(X, W, dn, preferred_element_type=jnp.float32)
    bias = (bv[...] + bd[...] + jnp.sum(bp[...], axis=0, keepdims=True))
    out[...] = acc + bias.reshape(1, 1, _EMBED)


@jax.jit
def kernel(value, depth, pos, Wv, bv, Wd, bd, Wp, bp):
    B, L = value.shape
    T = L // _S

    p0 = pos[:, :, 0]
    p1 = pos[:, :, 1]
    p2 = pos[:, :, 2]

    C = _COLS_PER_BLOCK
    grid = (L // C,)

    x_spec = pl.BlockSpec((B, C), lambda i: (0, i))

    out = pl.pallas_call(
        _embed_body,
        grid=grid,
        in_specs=[
            x_spec, x_spec, x_spec, x_spec, x_spec,
            pl.BlockSpec((_EMBED, _S), lambda i: (0, 0)),
            pl.BlockSpec((_EMBED, _S), lambda i: (0, 0)),
            pl.BlockSpec((3, _EMBED, _S), lambda i: (0, 0, 0)),
            pl.BlockSpec((1, _EMBED), lambda i: (0, 0)),
            pl.BlockSpec((1, _EMBED), lambda i: (0, 0)),
            pl.BlockSpec((3, _EMBED), lambda i: (0, 0)),
        ],
        out_specs=pl.BlockSpec((B, C // _S, _EMBED), lambda i: (0, i, 0)),
        out_shape=jax.ShapeDtypeStruct((B, T, _EMBED), jnp.float32),
    )(value, depth, p0, p1, p2, Wv, Wd, Wp,
      bv.reshape(1, _EMBED), bd.reshape(1, _EMBED), bp)

    return out


# speedup vs baseline: 1.0702x; 1.0702x over previous
import jax
import jax.numpy as jnp
from jax.experimental import pallas as pl

_EMBED = 256
_S = 8
_COLS_PER_BLOCK = 2048


def _embed_body(xv, xd, x0, x1, x2, Wv, Wd, Wp, bv, bd, bp, out):
    Bb, C = xv.shape
    Tb = C // _S
    xs = [x[...].astype(jnp.int8).reshape(Bb, Tb, _S)
          for x in (xv, xd, x0, x1, x2)]
    X = jnp.concatenate(xs, axis=2).astype(jnp.bfloat16)
    W = jnp.concatenate(
        [Wv[...], Wd[...], Wp[0], Wp[1], Wp[2]], axis=1).astype(jnp.bfloat16)
    dn = (((2,), (1,)), ((), ()))
    acc = jax.lax.dot_general(X, W, dn, preferred_element_type=jnp.float32)
    bias = (bv[...] + bd[...] + jnp.sum(bp[...], axis=0, keepdims=True))
    out[...] = acc + bias.reshape(1, 1, _EMBED)


@jax.jit
def kernel(value, depth, pos, Wv, bv, Wd, bd, Wp, bp):
    B, L = value.shape
    T = L // _S

    p0 = pos[:, :, 0]
    p1 = pos[:, :, 1]
    p2 = pos[:, :, 2]

    C = _COLS_PER_BLOCK
    grid = (L // C,)

    x_spec = pl.BlockSpec((B, C), lambda i: (0, i))

    out = pl.pallas_call(
        _embed_body,
        grid=grid,
        in_specs=[
            x_spec, x_spec, x_spec, x_spec, x_spec,
            pl.BlockSpec((_EMBED, _S), lambda i: (0, 0)),
            pl.BlockSpec((_EMBED, _S), lambda i: (0, 0)),
            pl.BlockSpec((3, _EMBED, _S), lambda i: (0, 0, 0)),
            pl.BlockSpec((1, _EMBED), lambda i: (0, 0)),
            pl.BlockSpec((1, _EMBED), lambda i: (0, 0)),
            pl.BlockSpec((3, _EMBED), lambda i: (0, 0)),
        ],
        out_specs=pl.BlockSpec((B, C // _S, _EMBED), lambda i: (0, i, 0)),
        out_shape=jax.ShapeDtypeStruct((B, T, _EMBED), jnp.float32),
    )(value, depth, p0, p1, p2, Wv, Wd, Wp,
      bv.reshape(1, _EMBED), bd.reshape(1, _EMBED), bp)

    return out
